# baseline (speedup 1.0000x reference)
import jax
import jax.numpy as jnp
from jax import lax
from jax.experimental import pallas as pl
from jax.experimental.pallas import tpu as pltpu
from jax.experimental.pallas import tpu_sc as plsc

T = 2048
D = 1024
H = 1024
E = 8
EL = 16
NEG_SLOPE = 0.01

B = 512
NBLK = (T * 2) // B + E - 1
NSH = T // B
SH_OFF = NBLK * B

NC = 2
NS = 16
NW = NC * NS
TPT = T // NW
L = 16


def _router_body(x_ref, Wr_ref, br_ref, noise_ref, w_ref,
                 rtr_ref, starts_ref, be_ref, nblk_ref):
    xf = x_ref[...]
    logits = (jnp.dot(xf, Wr_ref[...], preferred_element_type=jnp.float32)
              + br_ref[...] + noise_ref[...])
    lane = lax.broadcasted_iota(jnp.int32, (T, EL), 1)
    logits = jnp.where(lane < E, logits, -jnp.inf)
    v1 = jnp.max(logits, axis=-1, keepdims=True)
    i1 = jnp.min(jnp.where(logits == v1, lane, EL), axis=-1, keepdims=True)
    m1 = lane == i1
    l2 = jnp.where(m1, -jnp.inf, logits)
    v2 = jnp.max(l2, axis=-1, keepdims=True)
    i2 = jnp.min(jnp.where(l2 == v2, lane, EL), axis=-1, keepdims=True)
    m2 = lane == i2
    e2v = jnp.exp(v2 - v1)
    denom = 1.0 + e2v
    g1 = w_ref[1] / denom
    g2 = w_ref[1] * e2v / denom
    rtr_ref[...] = (jnp.where(lane == 0, i1.astype(jnp.float32), 0.0)
                    + jnp.where(lane == 1, i2.astype(jnp.float32), 0.0)
                    + jnp.where(lane == 2, g1, 0.0)
                    + jnp.where(lane == 3, g2, 0.0))
    oh = m1.astype(jnp.float32) + m2.astype(jnp.float32)
    ci = lax.broadcasted_iota(jnp.int32, (NW, T), 0)
    ti = lax.broadcasted_iota(jnp.int32, (NW, T), 1)
    csel = (ti // TPT == ci).astype(jnp.float32)
    histf = jnp.dot(csel, oh, preferred_element_type=jnp.float32)
    totf = jnp.sum(histf, axis=0, keepdims=True)
    nbf = jnp.floor((totf + (B - 1)) * (1.0 / B))
    ei = lax.broadcasted_iota(jnp.int32, (EL, EL), 0)
    ej = lax.broadcasted_iota(jnp.int32, (EL, EL), 1)
    ut = (ei <= ej).astype(jnp.float32)
    cum = jnp.dot(nbf, ut, preferred_element_type=jnp.float32)
    pstart = B * (cum - nbf)
    nblk_ref[...] = cum[:, E - 1:E].astype(jnp.int32)
    wi = lax.broadcasted_iota(jnp.int32, (NW, NW), 0)
    wj = lax.broadcasted_iota(jnp.int32, (NW, NW), 1)
    lt = (wj < wi).astype(jnp.float32)
    prefix = jnp.dot(lt, histf, preferred_element_type=jnp.float32)
    starts_ref[...] = (prefix + pstart).astype(jnp.int32)
    bi = lax.broadcasted_iota(jnp.int32, (16, EL), 0).astype(jnp.float32)
    bemask = jnp.where(lane[:16, :] < E, (cum <= bi).astype(jnp.float32), 0.0)
    be = jnp.sum(bemask, axis=-1, keepdims=True)
    be_ref[...] = jnp.minimum(be, float(E - 1)).astype(jnp.int32)


def _router(x, Wr16, br16, noise16, w):
    full = lambda shape: pl.BlockSpec(shape, lambda: (0,) * len(shape))
    return pl.pallas_call(
        _router_body,
        in_specs=[full((T, D)), full((D, EL)), full((1, EL)), full((T, EL)),
                  pl.BlockSpec(memory_space=pltpu.SMEM)],
        out_specs=[full((T, EL)), full((NW, EL)), full((16, 1)),
                   full((1, 1))],
        out_shape=[
            jax.ShapeDtypeStruct((T, EL), jnp.float32),
            jax.ShapeDtypeStruct((NW, EL), jnp.int32),
            jax.ShapeDtypeStruct((16, 1), jnp.int32),
            jax.ShapeDtypeStruct((1, 1), jnp.int32),
        ],
    )(x, Wr16, br16, noise16, w)


def _dispatch_body(x_hbm, e1_hbm, e2_hbm, starts_hbm,
                   xs_hbm, inv1_hbm, inv2_hbm,
                   x_v, e1_v, e2_v, srow_v, pos1_v, pos2_v, sem1, sem2):
    wid = lax.axis_index("s") * NC + lax.axis_index("c")
    base = wid * TPT
    pltpu.sync_copy(x_hbm.at[pl.ds(base, TPT)], x_v)
    pltpu.sync_copy(e1_hbm.at[pl.ds(base, TPT)], e1_v)
    pltpu.sync_copy(e2_hbm.at[pl.ds(base, TPT)], e2_v)
    pltpu.sync_copy(starts_hbm.at[pl.ds(wid, 1)], srow_v)
    srow = srow_v[0, :]
    elane = lax.iota(jnp.int32, L)

    zero = jnp.zeros((L,), jnp.int32)
    for k in range(TPT // L):
        pos1_v[pl.ds(k * L, L)] = zero
        pos2_v[pl.ds(k * L, L)] = zero

    for e in range(E):
        carry = jnp.sum(jnp.where(elane == e, srow, 0))
        for ev, pv in ((e1_v, pos1_v), (e2_v, pos2_v)):
            for k in range(TPT // L):
                ch = ev[pl.ds(k * L, L)]
                m = (ch == e).astype(jnp.int32)
                pc = plsc.cumsum(m)
                pos = (pc - m) + carry
                pv[pl.ds(k * L, L)] = pv[pl.ds(k * L, L)] + m * pos
                carry = carry + jnp.sum(m)

    pltpu.sync_copy(pos1_v, inv1_hbm.at[pl.ds(base, TPT)])
    pltpu.sync_copy(pos2_v, inv2_hbm.at[pl.ds(base, TPT)])
    c1 = pltpu.async_copy(x_v, xs_hbm.at[pos1_v], sem1)
    c2 = pltpu.async_copy(x_v, xs_hbm.at[pos2_v], sem2)
    c1.wait()
    c2.wait()


def _dispatch(x, e1, e2, starts):
    mesh = plsc.VectorSubcoreMesh(core_axis_name="c", subcore_axis_name="s")
    return pl.kernel(
        _dispatch_body,
        out_type=[
            jax.ShapeDtypeStruct((NBLK * B, D), jnp.float32),
            jax.ShapeDtypeStruct((T,), jnp.int32),
            jax.ShapeDtypeStruct((T,), jnp.int32),
        ],
        mesh=mesh,
        scratch_types=[
            pltpu.VMEM((TPT, D), jnp.float32),
            pltpu.VMEM((TPT,), jnp.int32),
            pltpu.VMEM((TPT,), jnp.int32),
            pltpu.VMEM((1, EL), jnp.int32),
            pltpu.VMEM((TPT,), jnp.int32),
            pltpu.VMEM((TPT,), jnp.int32),
            pltpu.SemaphoreType.DMA,
            pltpu.SemaphoreType.DMA,
        ],
        compiler_params=pltpu.CompilerParams(needs_layout_passes=False),
    )(x, e1, e2, starts)


def _experts_body(be_s, nblk_s, wv_s, xs_ref, x_ref, W1_ref, b1_ref, W2_ref,
                  b2_ref, sW1_ref, sb1_ref, sW2_ref, sb2_ref, ys_ref):
    b = pl.program_id(0)

    @pl.when(jnp.logical_and(b < NBLK, b < nblk_s[0]))
    def _expert():
        xsb = xs_ref[...].astype(jnp.bfloat16)
        h = (jnp.dot(xsb, W1_ref[0].astype(jnp.bfloat16),
                     preferred_element_type=jnp.float32) + b1_ref[0])
        h = jnp.where(h > 0, h, NEG_SLOPE * h).astype(jnp.bfloat16)
        ys_ref[...] = (jnp.dot(h, W2_ref[0].astype(jnp.bfloat16),
                               preferred_element_type=jnp.float32)
                       + b2_ref[0])

    @pl.when(b >= NBLK)
    def _shared():
        xb = x_ref[...].astype(jnp.bfloat16)
        h = (jnp.dot(xb, sW1_ref[...].astype(jnp.bfloat16),
                     preferred_element_type=jnp.float32) + sb1_ref[...])
        h = jnp.where(h > 0, h, NEG_SLOPE * h).astype(jnp.bfloat16)
        y = (jnp.dot(h, sW2_ref[...].astype(jnp.bfloat16),
                     preferred_element_type=jnp.float32) + sb2_ref[...])
        ys_ref[...] = wv_s[0] * y


def _experts(be, nblk, wv, xs, x, W1, b1, W2, b2, sW1, sb1, sW2, sb2):
    full = lambda shape: pl.BlockSpec(
        shape, lambda b, be, nb, wv: (0,) * len(shape))
    wmap = lambda b, be, nb, wv: (be[jnp.minimum(b, NBLK)], 0, 0)
    grid_spec = pltpu.PrefetchScalarGridSpec(
        num_scalar_prefetch=3,
        grid=(NBLK + NSH,),
        in_specs=[
            pl.BlockSpec((B, D),
                         lambda b, be, nb, wv: (jnp.minimum(b, NBLK - 1), 0)),
            pl.BlockSpec((B, D),
                         lambda b, be, nb, wv: (jnp.clip(b - NBLK, 0, NSH - 1), 0)),
            pl.BlockSpec((1, D, H), wmap),
            pl.BlockSpec((1, 1, H), wmap),
            pl.BlockSpec((1, H, D), wmap),
            pl.BlockSpec((1, 1, D), wmap),
            full((D, H)),
            full((1, H)),
            full((H, D)),
            full((1, D)),
        ],
        out_specs=pl.BlockSpec((B, D), lambda b, be, nb, wv: (b, 0)),
    )
    return pl.pallas_call(
        _experts_body,
        grid_spec=grid_spec,
        out_shape=jax.ShapeDtypeStruct(((NBLK + NSH) * B, D), jnp.float32),
        compiler_params=pltpu.CompilerParams(
            dimension_semantics=("arbitrary",)),
    )(be, nblk, wv, xs, x, W1, b1.reshape(E, 1, H), W2, b2.reshape(E, 1, D),
      sW1, sb1.reshape(1, H), sW2, sb2.reshape(1, D))


HALF = TPT // 2


def _combine_body(ys_hbm, g1_hbm, g2_hbm, inv1_hbm, inv2_hbm, out_hbm,
                  g1_v, g2_v, idx1_v, idx2_v, r1_v, r2_v, sh_v, sem1, sem2):
    wid = lax.axis_index("s") * NC + lax.axis_index("c")
    base = wid * TPT
    pltpu.sync_copy(g1_hbm.at[pl.ds(base, TPT)], g1_v)
    pltpu.sync_copy(g2_hbm.at[pl.ds(base, TPT)], g2_v)
    pltpu.sync_copy(inv1_hbm.at[pl.ds(base, TPT)], idx1_v)
    pltpu.sync_copy(inv2_hbm.at[pl.ds(base, TPT)], idx2_v)

    for hlf in range(2):
        c1 = pltpu.async_copy(ys_hbm.at[idx1_v.at[pl.ds(hlf * HALF, HALF)]],
                              r1_v, sem1)
        c2 = pltpu.async_copy(ys_hbm.at[idx2_v.at[pl.ds(hlf * HALF, HALF)]],
                              r2_v, sem2)
        pltpu.sync_copy(ys_hbm.at[pl.ds(SH_OFF + base + hlf * HALF, HALF)],
                        sh_v)
        c1.wait()
        c2.wait()

        def token(i, _):
            tok = hlf * HALF + i
            isplat = jnp.zeros((L,), jnp.int32) + tok
            g1 = plsc.load_gather(g1_v, [isplat])
            g2 = plsc.load_gather(g2_v, [isplat])

            def chunk(j, _):
                sl = pl.ds(j * L, L)
                r1_v[i, sl] = (g1 * r1_v[i, sl] + g2 * r2_v[i, sl]
                               + sh_v[i, sl])
                return 0
            lax.fori_loop(0, D // L, chunk, 0)
            return 0
        lax.fori_loop(0, HALF, token, 0)
        pltpu.sync_copy(r1_v, out_hbm.at[pl.ds(base + hlf * HALF, HALF)])


def _combine(ys, g1, g2, inv1, inv2):
    mesh = plsc.VectorSubcoreMesh(core_axis_name="c", subcore_axis_name="s")
    return pl.kernel(
        _combine_body,
        out_type=jax.ShapeDtypeStruct((T, D), jnp.float32),
        mesh=mesh,
        scratch_types=[
            pltpu.VMEM((TPT,), jnp.float32),
            pltpu.VMEM((TPT,), jnp.float32),
            pltpu.VMEM((TPT,), jnp.int32),
            pltpu.VMEM((TPT,), jnp.int32),
            pltpu.VMEM((HALF, D), jnp.float32),
            pltpu.VMEM((HALF, D), jnp.float32),
            pltpu.VMEM((HALF, D), jnp.float32),
            pltpu.SemaphoreType.DMA,
            pltpu.SemaphoreType.DMA,
        ],
        compiler_params=pltpu.CompilerParams(needs_layout_passes=False),
    )(ys, g1, g2, inv1, inv2)


@jax.jit
def kernel(x, Wr, br, W1, b1, W2, b2, sW1, sb1, sW2, sb2, alpha, beta, noise):
    w = jax.nn.softmax(jnp.stack([alpha, beta]))
    pad = ((0, 0), (0, EL - E))
    rtr, starts, be, nblk = _router(
        x, jnp.pad(Wr, pad), jnp.pad(br.reshape(1, E), pad),
        jnp.pad(noise, pad), w)
    e1 = rtr[:, 0].astype(jnp.int32)
    e2 = rtr[:, 1].astype(jnp.int32)
    xs, inv1, inv2 = _dispatch(x, e1, e2, starts)
    ys = _experts(be.reshape(16), nblk.reshape(1), w, xs, x,
                  W1, b1, W2, b2, sW1, sb1, sW2, sb2)
    return _combine(ys, rtr[:, 2], rtr[:, 3], inv1, inv2)


# speedup vs baseline: 1.1170x; 1.1170x over previous
import jax
import jax.numpy as jnp
from jax import lax
from jax.experimental import pallas as pl
from jax.experimental.pallas import tpu as pltpu
from jax.experimental.pallas import tpu_sc as plsc

T = 2048
D = 1024
H = 1024
E = 8
EL = 16
NEG_SLOPE = 0.01

B = 512
NBLK = (T * 2) // B + E - 1
NSH = T // B
SH_OFF = NBLK * B

NC = 2
NS = 16
NW = NC * NS
TPT = T // NW
L = 16


def _router_body(x_ref, Wr_ref, br_ref, noise_ref, w_ref,
                 rtr_ref, starts_ref, be_ref, nblk_ref):
    xf = x_ref[...]
    logits = (jnp.dot(xf, Wr_ref[...], preferred_element_type=jnp.float32)
              + br_ref[...] + noise_ref[...])
    lane = lax.broadcasted_iota(jnp.int32, (T, EL), 1)
    logits = jnp.where(lane < E, logits, -jnp.inf)
    v1 = jnp.max(logits, axis=-1, keepdims=True)
    i1 = jnp.min(jnp.where(logits == v1, lane, EL), axis=-1, keepdims=True)
    m1 = lane == i1
    l2 = jnp.where(m1, -jnp.inf, logits)
    v2 = jnp.max(l2, axis=-1, keepdims=True)
    i2 = jnp.min(jnp.where(l2 == v2, lane, EL), axis=-1, keepdims=True)
    m2 = lane == i2
    e2v = jnp.exp(v2 - v1)
    denom = 1.0 + e2v
    g1 = w_ref[1] / denom
    g2 = w_ref[1] * e2v / denom
    rtr_ref[...] = (jnp.where(lane == 0, i1.astype(jnp.float32), 0.0)
                    + jnp.where(lane == 1, i2.astype(jnp.float32), 0.0)
                    + jnp.where(lane == 2, g1, 0.0)
                    + jnp.where(lane == 3, g2, 0.0))
    oh = m1.astype(jnp.float32) + m2.astype(jnp.float32)
    ci = lax.broadcasted_iota(jnp.int32, (NW, T), 0)
    ti = lax.broadcasted_iota(jnp.int32, (NW, T), 1)
    csel = (ti // TPT == ci).astype(jnp.float32)
    histf = jnp.dot(csel, oh, preferred_element_type=jnp.float32)
    totf = jnp.sum(histf, axis=0, keepdims=True)
    nbf = jnp.floor((totf + (B - 1)) * (1.0 / B))
    ei = lax.broadcasted_iota(jnp.int32, (EL, EL), 0)
    ej = lax.broadcasted_iota(jnp.int32, (EL, EL), 1)
    ut = (ei <= ej).astype(jnp.float32)
    cum = jnp.dot(nbf, ut, preferred_element_type=jnp.float32)
    pstart = B * (cum - nbf)
    nblk_ref[...] = cum[:, E - 1:E].astype(jnp.int32)
    wi = lax.broadcasted_iota(jnp.int32, (NW, NW), 0)
    wj = lax.broadcasted_iota(jnp.int32, (NW, NW), 1)
    lt = (wj < wi).astype(jnp.float32)
    prefix = jnp.dot(lt, histf, preferred_element_type=jnp.float32)
    starts_ref[...] = (prefix + pstart).astype(jnp.int32)
    bi = lax.broadcasted_iota(jnp.int32, (16, EL), 0).astype(jnp.float32)
    bemask = jnp.where(lane[:16, :] < E, (cum <= bi).astype(jnp.float32), 0.0)
    be = jnp.sum(bemask, axis=-1, keepdims=True)
    be_ref[...] = jnp.minimum(be, float(E - 1)).astype(jnp.int32)


def _router(x, Wr16, br16, noise16, w):
    full = lambda shape: pl.BlockSpec(shape, lambda: (0,) * len(shape))
    return pl.pallas_call(
        _router_body,
        in_specs=[full((T, D)), full((D, EL)), full((1, EL)), full((T, EL)),
                  pl.BlockSpec(memory_space=pltpu.SMEM)],
        out_specs=[full((T, EL)), full((NW, EL)), full((16, 1)),
                   full((1, 1))],
        out_shape=[
            jax.ShapeDtypeStruct((T, EL), jnp.float32),
            jax.ShapeDtypeStruct((NW, EL), jnp.int32),
            jax.ShapeDtypeStruct((16, 1), jnp.int32),
            jax.ShapeDtypeStruct((1, 1), jnp.int32),
        ],
    )(x, Wr16, br16, noise16, w)


def _dispatch_body(x_hbm, e1_hbm, e2_hbm, starts_hbm,
                   xs_hbm, inv1_hbm, inv2_hbm,
                   x_v, e1_v, e2_v, srow_v, pos1_v, pos2_v, sem1, sem2):
    wid = lax.axis_index("s") * NC + lax.axis_index("c")
    base = wid * TPT
    pltpu.sync_copy(x_hbm.at[pl.ds(base, TPT)], x_v)
    pltpu.sync_copy(e1_hbm.at[pl.ds(base, TPT)], e1_v)
    pltpu.sync_copy(e2_hbm.at[pl.ds(base, TPT)], e2_v)
    pltpu.sync_copy(starts_hbm.at[pl.ds(wid, 1)], srow_v)
    srow = srow_v[0, :]
    elane = lax.iota(jnp.int32, L)

    zero = jnp.zeros((L,), jnp.int32)
    for k in range(TPT // L):
        pos1_v[pl.ds(k * L, L)] = zero
        pos2_v[pl.ds(k * L, L)] = zero

    for e in range(E):
        carry = jnp.sum(jnp.where(elane == e, srow, 0))
        for ev, pv in ((e1_v, pos1_v), (e2_v, pos2_v)):
            for k in range(TPT // L):
                ch = ev[pl.ds(k * L, L)]
                m = (ch == e).astype(jnp.int32)
                pc = plsc.cumsum(m)
                pos = (pc - m) + carry
                pv[pl.ds(k * L, L)] = pv[pl.ds(k * L, L)] + m * pos
                carry = carry + jnp.sum(m)

    pltpu.sync_copy(pos1_v, inv1_hbm.at[pl.ds(base, TPT)])
    pltpu.sync_copy(pos2_v, inv2_hbm.at[pl.ds(base, TPT)])
    c1 = pltpu.async_copy(x_v, xs_hbm.at[pos1_v], sem1)
    c2 = pltpu.async_copy(x_v, xs_hbm.at[pos2_v], sem2)
    c1.wait()
    c2.wait()


def _dispatch(x, e1, e2, starts):
    mesh = plsc.VectorSubcoreMesh(core_axis_name="c", subcore_axis_name="s")
    return pl.kernel(
        _dispatch_body,
        out_type=[
            jax.ShapeDtypeStruct((NBLK * B, D), jnp.float32),
            jax.ShapeDtypeStruct((T,), jnp.int32),
            jax.ShapeDtypeStruct((T,), jnp.int32),
        ],
        mesh=mesh,
        scratch_types=[
            pltpu.VMEM((TPT, D), jnp.float32),
            pltpu.VMEM((TPT,), jnp.int32),
            pltpu.VMEM((TPT,), jnp.int32),
            pltpu.VMEM((1, EL), jnp.int32),
            pltpu.VMEM((TPT,), jnp.int32),
            pltpu.VMEM((TPT,), jnp.int32),
            pltpu.SemaphoreType.DMA,
            pltpu.SemaphoreType.DMA,
        ],
        compiler_params=pltpu.CompilerParams(needs_layout_passes=False),
    )(x, e1, e2, starts)


def _experts_body(be_s, nblk_s, wv_s, xs_ref, x_ref, W1_ref, b1_ref, W2_ref,
                  b2_ref, sW1_ref, sb1_ref, sW2_ref, sb2_ref, ys_ref):
    b = pl.program_id(0)

    @pl.when(jnp.logical_and(b < NBLK, b < nblk_s[0]))
    def _expert():
        h = (jnp.dot(xs_ref[...].astype(jnp.bfloat16),
                     W1_ref[0].astype(jnp.bfloat16),
                     preferred_element_type=jnp.float32) + b1_ref[0])
        h = jnp.where(h > 0, h, NEG_SLOPE * h).astype(jnp.bfloat16)
        ys_ref[...] = (jnp.dot(h, W2_ref[0].astype(jnp.bfloat16),
                               preferred_element_type=jnp.float32)
                       + b2_ref[0])

    @pl.when(b >= NBLK)
    def _shared():
        h = (jnp.dot(x_ref[...].astype(jnp.bfloat16),
                     sW1_ref[...].astype(jnp.bfloat16),
                     preferred_element_type=jnp.float32) + sb1_ref[...])
        h = jnp.where(h > 0, h, NEG_SLOPE * h).astype(jnp.bfloat16)
        y = (jnp.dot(h, sW2_ref[...].astype(jnp.bfloat16),
                     preferred_element_type=jnp.float32) + sb2_ref[...])
        ys_ref[...] = wv_s[0] * y


def _experts(be, nblk, wv, xs, x, W1, b1, W2, b2, sW1, sb1, sW2, sb2):
    full = lambda shape: pl.BlockSpec(
        shape, lambda b, be, nb, wv: (0,) * len(shape))
    wmap = lambda b, be, nb, wv: (be[jnp.minimum(b, NBLK)], 0, 0)
    grid_spec = pltpu.PrefetchScalarGridSpec(
        num_scalar_prefetch=3,
        grid=(NBLK + NSH,),
        in_specs=[
            pl.BlockSpec((B, D),
                         lambda b, be, nb, wv: (jnp.minimum(b, NBLK - 1), 0)),
            pl.BlockSpec((B, D),
                         lambda b, be, nb, wv: (jnp.clip(b - NBLK, 0, NSH - 1), 0)),
            pl.BlockSpec((1, D, H), wmap),
            pl.BlockSpec((1, 1, H), wmap),
            pl.BlockSpec((1, H, D), wmap),
            pl.BlockSpec((1, 1, D), wmap),
            full((D, H)),
            full((1, H)),
            full((H, D)),
            full((1, D)),
        ],
        out_specs=pl.BlockSpec((B, D), lambda b, be, nb, wv: (b, 0)),
    )
    return pl.pallas_call(
        _experts_body,
        grid_spec=grid_spec,
        out_shape=jax.ShapeDtypeStruct(((NBLK + NSH) * B, D), jnp.float32),
        compiler_params=pltpu.CompilerParams(
            dimension_semantics=("arbitrary",)),
    )(be, nblk, wv, xs, x, W1, b1.reshape(E, 1, H), W2, b2.reshape(E, 1, D),
      sW1, sb1.reshape(1, H), sW2, sb2.reshape(1, D))


HALF = TPT // 2


def _gather2_body(ys_hbm, inv1_hbm, inv2_hbm, r1_hbm, r2_hbm,
                  idx1_v, idx2_v, r1_v, r2_v, sem1, sem2):
    wid = lax.axis_index("s") * NC + lax.axis_index("c")
    base = wid * TPT
    pltpu.sync_copy(inv1_hbm.at[pl.ds(base, TPT)], idx1_v)
    pltpu.sync_copy(inv2_hbm.at[pl.ds(base, TPT)], idx2_v)
    for hlf in range(2):
        c1 = pltpu.async_copy(ys_hbm.at[idx1_v.at[pl.ds(hlf * HALF, HALF)]],
                              r1_v, sem1)
        c2 = pltpu.async_copy(ys_hbm.at[idx2_v.at[pl.ds(hlf * HALF, HALF)]],
                              r2_v, sem2)
        c1.wait()
        pltpu.sync_copy(r1_v, r1_hbm.at[pl.ds(base + hlf * HALF, HALF)])
        c2.wait()
        pltpu.sync_copy(r2_v, r2_hbm.at[pl.ds(base + hlf * HALF, HALF)])


def _gather2(ys, inv1, inv2):
    mesh = plsc.VectorSubcoreMesh(core_axis_name="c", subcore_axis_name="s")
    return pl.kernel(
        _gather2_body,
        out_type=[
            jax.ShapeDtypeStruct((T, D), jnp.float32),
            jax.ShapeDtypeStruct((T, D), jnp.float32),
        ],
        mesh=mesh,
        scratch_types=[
            pltpu.VMEM((TPT,), jnp.int32),
            pltpu.VMEM((TPT,), jnp.int32),
            pltpu.VMEM((HALF, D), jnp.float32),
            pltpu.VMEM((HALF, D), jnp.float32),
            pltpu.SemaphoreType.DMA,
            pltpu.SemaphoreType.DMA,
        ],
        compiler_params=pltpu.CompilerParams(needs_layout_passes=False),
    )(ys, inv1, inv2)


def _final_body(g1_ref, g2_ref, r1_ref, r2_ref, ys_ref, out_ref):
    out_ref[...] = (g1_ref[...] * r1_ref[...] + g2_ref[...] * r2_ref[...]
                    + ys_ref[...])


def _final(g1c, g2c, r1, r2, ys):
    tb = lambda: pl.BlockSpec((B, D), lambda i: (i, 0))
    return pl.pallas_call(
        _final_body,
        grid=(T // B,),
        in_specs=[
            pl.BlockSpec((B, 1), lambda i: (i, 0)),
            pl.BlockSpec((B, 1), lambda i: (i, 0)),
            tb(),
            tb(),
            pl.BlockSpec((B, D), lambda i: (SH_OFF // B + i, 0)),
        ],
        out_specs=tb(),
        out_shape=jax.ShapeDtypeStruct((T, D), jnp.float32),
        compiler_params=pltpu.CompilerParams(
            dimension_semantics=("arbitrary",)),
    )(g1c, g2c, r1, r2, ys)


@jax.jit
def kernel(x, Wr, br, W1, b1, W2, b2, sW1, sb1, sW2, sb2, alpha, beta, noise):
    w = jax.nn.softmax(jnp.stack([alpha, beta]))
    pad = ((0, 0), (0, EL - E))
    rtr, starts, be, nblk = _router(
        x, jnp.pad(Wr, pad), jnp.pad(br.reshape(1, E), pad),
        jnp.pad(noise, pad), w)
    e1 = rtr[:, 0].astype(jnp.int32)
    e2 = rtr[:, 1].astype(jnp.int32)
    xs, inv1, inv2 = _dispatch(x, e1, e2, starts)
    ys = _experts(be.reshape(16), nblk.reshape(1), w, xs, x,
                  W1, b1, W2, b2, sW1, sb1, sW2, sb2)
    r1, r2 = _gather2(ys, inv1, inv2)
    return _final(rtr[:, 2:3], rtr[:, 3:4], r1, r2, ys)


# speedup vs baseline: 1.3300x; 1.1907x over previous
import jax
import jax.numpy as jnp
from jax import lax
from jax.experimental import pallas as pl
from jax.experimental.pallas import tpu as pltpu
from jax.experimental.pallas import tpu_sc as plsc

T = 2048
D = 1024
H = 1024
E = 8
EL = 16
NEG_SLOPE = 0.01

B = 512
NBLK = (T * 2) // B + E - 1
NSH = T // B
SH_OFF = NBLK * B

NC = 2
NS = 16
NW = NC * NS
TPT = T // NW
L = 16


def _rne16(v):
    return v + 0x7FFF + ((v >> 16) & 1)


def _pack(y):
    h = y.shape[1] // 2
    b = jax.lax.bitcast_convert_type(y, jnp.int32)
    lo = (_rne16(b[:, :h]) >> 16) & 0xFFFF
    hi = _rne16(b[:, h:]) & jnp.int32(-65536)
    return lo | hi


def _unpack(p):
    a = jax.lax.bitcast_convert_type(p << 16, jnp.float32)
    c = jax.lax.bitcast_convert_type(p & jnp.int32(-65536), jnp.float32)
    return jnp.concatenate([a, c], axis=1)


def _router_body(x_ref, Wr_ref, br_ref, noise_ref, w_ref,
                 rtr_ref, starts_ref, be_ref, nblk_ref, xbp_ref):
    xf = x_ref[...]
    xbp_ref[...] = _pack(xf)
    l8 = (jnp.dot(xf, Wr_ref[...], preferred_element_type=jnp.float32)
          + br_ref[...] + noise_ref[...])
    logits = jnp.concatenate(
        [l8, jnp.full((T, EL - E), -jnp.inf, jnp.float32)], axis=1)
    lane = lax.broadcasted_iota(jnp.int32, (T, EL), 1)
    v1 = jnp.max(logits, axis=-1, keepdims=True)
    i1 = jnp.min(jnp.where(logits == v1, lane, EL), axis=-1, keepdims=True)
    m1 = lane == i1
    l2 = jnp.where(m1, -jnp.inf, logits)
    v2 = jnp.max(l2, axis=-1, keepdims=True)
    i2 = jnp.min(jnp.where(l2 == v2, lane, EL), axis=-1, keepdims=True)
    m2 = lane == i2
    e2v = jnp.exp(v2 - v1)
    denom = 1.0 + e2v
    g1 = w_ref[1] / denom
    g2 = w_ref[1] * e2v / denom
    rtr_ref[...] = (jnp.where(lane == 0, i1.astype(jnp.float32), 0.0)
                    + jnp.where(lane == 1, i2.astype(jnp.float32), 0.0)
                    + jnp.where(lane == 2, g1, 0.0)
                    + jnp.where(lane == 3, g2, 0.0))
    oh = m1.astype(jnp.float32) + m2.astype(jnp.float32)
    ci = lax.broadcasted_iota(jnp.int32, (NW, T), 0)
    ti = lax.broadcasted_iota(jnp.int32, (NW, T), 1)
    csel = (ti // TPT == ci).astype(jnp.float32)
    histf = jnp.dot(csel, oh, preferred_element_type=jnp.float32)
    totf = jnp.sum(histf, axis=0, keepdims=True)
    nbf = jnp.floor((totf + (B - 1)) * (1.0 / B))
    ei = lax.broadcasted_iota(jnp.int32, (EL, EL), 0)
    ej = lax.broadcasted_iota(jnp.int32, (EL, EL), 1)
    ut = (ei <= ej).astype(jnp.float32)
    cum = jnp.dot(nbf, ut, preferred_element_type=jnp.float32)
    pstart = B * (cum - nbf)
    nblk_ref[...] = cum[:, E - 1:E].astype(jnp.int32)
    wi = lax.broadcasted_iota(jnp.int32, (NW, NW), 0)
    wj = lax.broadcasted_iota(jnp.int32, (NW, NW), 1)
    lt = (wj < wi).astype(jnp.float32)
    prefix = jnp.dot(lt, histf, preferred_element_type=jnp.float32)
    starts_ref[...] = (prefix + pstart).astype(jnp.int32)
    bi = lax.broadcasted_iota(jnp.int32, (16, EL), 0).astype(jnp.float32)
    bemask = jnp.where(lane[:16, :] < E, (cum <= bi).astype(jnp.float32), 0.0)
    be = jnp.sum(bemask, axis=-1, keepdims=True)
    be_ref[...] = jnp.minimum(be, float(E - 1)).astype(jnp.int32)


def _router(x, Wr, br, noise, w):
    full = lambda shape: pl.BlockSpec(shape, lambda: (0,) * len(shape))
    return pl.pallas_call(
        _router_body,
        in_specs=[full((T, D)), full((D, E)), full((1, E)), full((T, E)),
                  pl.BlockSpec(memory_space=pltpu.SMEM)],
        out_specs=[full((T, EL)), full((NW, EL)), full((16, 1)),
                   full((1, 1)), full((T, D // 2))],
        out_shape=[
            jax.ShapeDtypeStruct((T, EL), jnp.float32),
            jax.ShapeDtypeStruct((NW, EL), jnp.int32),
            jax.ShapeDtypeStruct((16, 1), jnp.int32),
            jax.ShapeDtypeStruct((1, 1), jnp.int32),
            jax.ShapeDtypeStruct((T, D // 2), jnp.int32),
        ],
    )(x, Wr, br, noise, w)


def _dispatch_body(x_hbm, e1_hbm, e2_hbm, starts_hbm,
                   xs_hbm, inv1_hbm, inv2_hbm,
                   x_v, e1_v, e2_v, srow_v, pos1_v, pos2_v, sem1, sem2):
    wid = lax.axis_index("s") * NC + lax.axis_index("c")
    base = wid * TPT
    pltpu.sync_copy(x_hbm.at[pl.ds(base, TPT)], x_v)
    pltpu.sync_copy(e1_hbm.at[pl.ds(base, TPT)], e1_v)
    pltpu.sync_copy(e2_hbm.at[pl.ds(base, TPT)], e2_v)
    pltpu.sync_copy(starts_hbm.at[pl.ds(wid, 1)], srow_v)
    srow = srow_v[0, :]
    elane = lax.iota(jnp.int32, L)

    zero = jnp.zeros((L,), jnp.int32)
    for k in range(TPT // L):
        pos1_v[pl.ds(k * L, L)] = zero
        pos2_v[pl.ds(k * L, L)] = zero

    for e in range(E):
        carry = jnp.sum(jnp.where(elane == e, srow, 0))
        for ev, pv in ((e1_v, pos1_v), (e2_v, pos2_v)):
            for k in range(TPT // L):
                ch = ev[pl.ds(k * L, L)]
                m = (ch == e).astype(jnp.int32)
                pc = plsc.cumsum(m)
                pos = (pc - m) + carry
                pv[pl.ds(k * L, L)] = pv[pl.ds(k * L, L)] + m * pos
                carry = carry + jnp.sum(m)

    pltpu.sync_copy(pos1_v, inv1_hbm.at[pl.ds(base, TPT)])
    pltpu.sync_copy(pos2_v, inv2_hbm.at[pl.ds(base, TPT)])
    c1 = pltpu.async_copy(x_v, xs_hbm.at[pos1_v], sem1)
    c2 = pltpu.async_copy(x_v, xs_hbm.at[pos2_v], sem2)
    c1.wait()
    c2.wait()


def _dispatch(x, e1, e2, starts):
    mesh = plsc.VectorSubcoreMesh(core_axis_name="c", subcore_axis_name="s")
    return pl.kernel(
        _dispatch_body,
        out_type=[
            jax.ShapeDtypeStruct((NBLK * B, D // 2), jnp.int32),
            jax.ShapeDtypeStruct((T,), jnp.int32),
            jax.ShapeDtypeStruct((T,), jnp.int32),
        ],
        mesh=mesh,
        scratch_types=[
            pltpu.VMEM((TPT, D // 2), jnp.int32),
            pltpu.VMEM((TPT,), jnp.int32),
            pltpu.VMEM((TPT,), jnp.int32),
            pltpu.VMEM((1, EL), jnp.int32),
            pltpu.VMEM((TPT,), jnp.int32),
            pltpu.VMEM((TPT,), jnp.int32),
            pltpu.SemaphoreType.DMA,
            pltpu.SemaphoreType.DMA,
        ],
        compiler_params=pltpu.CompilerParams(needs_layout_passes=False),
    )(x, e1, e2, starts)


def _experts_body(be_s, nblk_s, xs_ref, W1_ref, b1_ref, W2_ref,
                  b2_ref, ys_ref):
    b = pl.program_id(0)

    @pl.when(b < nblk_s[0])
    def _expert():
        h = (jnp.dot(_unpack(xs_ref[...]), W1_ref[0],
                     precision=lax.Precision.DEFAULT,
                     preferred_element_type=jnp.float32) + b1_ref[0])
        h = jnp.where(h > 0, h, NEG_SLOPE * h)
        y = (jnp.dot(h, W2_ref[0],
                     precision=lax.Precision.DEFAULT,
                     preferred_element_type=jnp.float32) + b2_ref[0])
        ys_ref[...] = _pack(y)


def _experts(be, nblk, xs, W1, b1, W2, b2):
    wmap = lambda b, be, nb: (be[jnp.minimum(b, NBLK)], 0, 0)
    grid_spec = pltpu.PrefetchScalarGridSpec(
        num_scalar_prefetch=2,
        grid=(NBLK,),
        in_specs=[
            pl.BlockSpec((B, D // 2), lambda b, be, nb: (b, 0)),
            pl.BlockSpec((1, D, H), wmap),
            pl.BlockSpec((1, 1, H), wmap),
            pl.BlockSpec((1, H, D), wmap),
            pl.BlockSpec((1, 1, D), wmap),
        ],
        out_specs=pl.BlockSpec((B, D // 2), lambda b, be, nb: (b, 0)),
    )
    return pl.pallas_call(
        _experts_body,
        grid_spec=grid_spec,
        out_shape=jax.ShapeDtypeStruct((NBLK * B, D // 2), jnp.int32),
        compiler_params=pltpu.CompilerParams(
            dimension_semantics=("arbitrary",)),
    )(be, nblk, xs, W1, b1.reshape(E, 1, H), W2, b2.reshape(E, 1, D))


def _shared_body(wv_s, x_ref, sW1_ref, sb1_ref, sW2_ref, sb2_ref, sh_ref):
    h = (jnp.dot(_unpack(x_ref[...]), sW1_ref[...],
                 precision=lax.Precision.DEFAULT,
                 preferred_element_type=jnp.float32) + sb1_ref[...])
    h = jnp.where(h > 0, h, NEG_SLOPE * h)
    y = (jnp.dot(h, sW2_ref[...],
                 precision=lax.Precision.DEFAULT,
                 preferred_element_type=jnp.float32) + sb2_ref[...])
    sh_ref[...] = _pack(wv_s[0] * y)


def _shared(wv, xbp, sW1, sb1, sW2, sb2):
    full = lambda shape: pl.BlockSpec(shape, lambda b, wv: (0,) * len(shape))
    grid_spec = pltpu.PrefetchScalarGridSpec(
        num_scalar_prefetch=1,
        grid=(NSH,),
        in_specs=[
            pl.BlockSpec((B, D // 2), lambda b, wv: (b, 0)),
            full((D, H)),
            full((1, H)),
            full((H, D)),
            full((1, D)),
        ],
        out_specs=pl.BlockSpec((B, D // 2), lambda b, wv: (b, 0)),
    )
    return pl.pallas_call(
        _shared_body,
        grid_spec=grid_spec,
        out_shape=jax.ShapeDtypeStruct((T, D // 2), jnp.int32),
        compiler_params=pltpu.CompilerParams(
            dimension_semantics=("arbitrary",)),
    )(wv, xbp, sW1, sb1.reshape(1, H), sW2, sb2.reshape(1, D))


HALF = TPT // 2


def _gather2_body(ys_hbm, inv1_hbm, inv2_hbm, r1_hbm, r2_hbm,
                  idx1_v, idx2_v, r1_v, r2_v, sem1, sem2):
    wid = lax.axis_index("s") * NC + lax.axis_index("c")
    base = wid * TPT
    pltpu.sync_copy(inv1_hbm.at[pl.ds(base, TPT)], idx1_v)
    pltpu.sync_copy(inv2_hbm.at[pl.ds(base, TPT)], idx2_v)
    c1 = pltpu.async_copy(ys_hbm.at[idx1_v], r1_v, sem1)
    c2 = pltpu.async_copy(ys_hbm.at[idx2_v], r2_v, sem2)
    c1.wait()
    pltpu.sync_copy(r1_v, r1_hbm.at[pl.ds(base, TPT)])
    c2.wait()
    pltpu.sync_copy(r2_v, r2_hbm.at[pl.ds(base, TPT)])


def _gather2(ys, inv1, inv2):
    mesh = plsc.VectorSubcoreMesh(core_axis_name="c", subcore_axis_name="s")
    return pl.kernel(
        _gather2_body,
        out_type=[
            jax.ShapeDtypeStruct((T, D // 2), jnp.int32),
            jax.ShapeDtypeStruct((T, D // 2), jnp.int32),
        ],
        mesh=mesh,
        scratch_types=[
            pltpu.VMEM((TPT,), jnp.int32),
            pltpu.VMEM((TPT,), jnp.int32),
            pltpu.VMEM((TPT, D // 2), jnp.int32),
            pltpu.VMEM((TPT, D // 2), jnp.int32),
            pltpu.SemaphoreType.DMA,
            pltpu.SemaphoreType.DMA,
        ],
        compiler_params=pltpu.CompilerParams(needs_layout_passes=False),
    )(ys, inv1, inv2)


def _final_body(g1_ref, g2_ref, r1_ref, r2_ref, ys_ref, out_ref):
    out_ref[...] = (g1_ref[...] * _unpack(r1_ref[...])
                    + g2_ref[...] * _unpack(r2_ref[...])
                    + _unpack(ys_ref[...]))


def _final(g1c, g2c, r1, r2, shp):
    tb = lambda: pl.BlockSpec((B, D // 2), lambda i: (i, 0))
    return pl.pallas_call(
        _final_body,
        grid=(T // B,),
        in_specs=[
            pl.BlockSpec((B, 1), lambda i: (i, 0)),
            pl.BlockSpec((B, 1), lambda i: (i, 0)),
            tb(),
            tb(),
            tb(),
        ],
        out_specs=pl.BlockSpec((B, D), lambda i: (i, 0)),
        out_shape=jax.ShapeDtypeStruct((T, D), jnp.float32),
        compiler_params=pltpu.CompilerParams(
            dimension_semantics=("arbitrary",)),
    )(g1c, g2c, r1, r2, shp)


@jax.jit
def kernel(x, Wr, br, W1, b1, W2, b2, sW1, sb1, sW2, sb2, alpha, beta, noise):
    w = jax.nn.softmax(jnp.stack([alpha, beta]))
    rtr, starts, be, nblk, xbp = _router(x, Wr, br.reshape(1, E), noise, w)
    e1 = rtr[:, 0].astype(jnp.int32)
    e2 = rtr[:, 1].astype(jnp.int32)
    xs, inv1, inv2 = _dispatch(xbp, e1, e2, starts)
    shp = _shared(w, xbp, sW1, sb1, sW2, sb2)
    ys = _experts(be.reshape(16), nblk.reshape(1), xs, W1, b1, W2, b2)
    r1, r2 = _gather2(ys, inv1, inv2)
    return _final(rtr[:, 2:3], rtr[:, 3:4], r1, r2, shp)
